# baseline (speedup 1.0000x reference)
import functools

import jax
import jax.numpy as jnp
from jax import lax
from jax.experimental import pallas as pl
from jax.experimental.pallas import tpu as pltpu
from jax.experimental.pallas import tpu_sc as plsc

DIM = 64
PAIR_BN = 16384
PAIR_SH = 14


def _pair_rows_tc(ent_t):
    NE = ent_t.shape[1]
    BN = PAIR_BN
    grid = pl.cdiv(NE, 2 * BN)

    def body(in_ref, out_ref):
        x = in_ref[...]
        out_ref[...] = jnp.concatenate(
            [x[:, :BN].T, x[:, BN:].T], axis=1)

    return pl.pallas_call(
        body,
        grid=(grid,),
        in_specs=[pl.BlockSpec((DIM, 2 * BN), lambda j: (0, j))],
        out_specs=pl.BlockSpec((BN, 2 * DIM), lambda j: (j, 0)),
        out_shape=jax.ShapeDtypeStruct((grid * BN, 2 * DIM), jnp.float32),
    )(ent_t)


def _transh_sc(h_ids, r_ids, t_ids, ent_pair, rn_table):
    B = h_ids.shape[0]
    NC, NS, L = 2, 16, 16
    NW = NC * NS
    PW = B // NW
    C = min(128, PW)
    NCH = PW // C
    G = C // L

    mesh = plsc.VectorSubcoreMesh(
        core_axis_name="c", subcore_axis_name="s", num_cores=NC, num_subcores=NS)

    @functools.partial(
        pl.kernel,
        mesh=mesh,
        out_type=jax.ShapeDtypeStruct((B,), jnp.float32),
        compiler_params=pltpu.CompilerParams(
            needs_layout_passes=False, use_tc_tiling_on_sc=True),
        scratch_types=[
            pltpu.VMEM((2, C), jnp.int32),
            pltpu.VMEM((2, C), jnp.int32),
            pltpu.VMEM((2, C), jnp.int32),
            pltpu.VMEM((2, C), jnp.int32),
            pltpu.VMEM((2, C), jnp.int32),
            pltpu.VMEM((2, C, 2 * DIM), jnp.float32),
            pltpu.VMEM((2, C, 2 * DIM), jnp.float32),
            pltpu.VMEM((2, C, 2 * DIM), jnp.float32),
            pltpu.VMEM((DIM, L), jnp.float32),
            pltpu.VMEM((DIM, L), jnp.float32),
            pltpu.VMEM((PW,), jnp.float32),
            pltpu.SemaphoreType.DMA,
            pltpu.SemaphoreType.DMA,
        ],
    )
    def _k(h_hbm, r_hbm, t_hbm, ent_hbm, rn_hbm, out_hbm,
           hids, tids, hrow, trow, rids, hrows, trows, rn_rows,
           u_scr, n_scr, outv, sem0, sem1):
        wid = lax.axis_index("s") * NC + lax.axis_index("c")
        lane = lax.iota(jnp.int32, L)
        sems = (sem0, sem1)

        def fire(ch, s):
            base = wid * PW + ch * C
            pltpu.sync_copy(h_hbm.at[pl.ds(base, C)], hids.at[s])
            pltpu.sync_copy(t_hbm.at[pl.ds(base, C)], tids.at[s])
            pltpu.sync_copy(r_hbm.at[pl.ds(base, C)], rids.at[s])

            @pl.loop(0, G)
            def _halve(i):
                sl = pl.ds(i * L, L)
                hv = hids[s, sl]
                tv = tids[s, sl]
                hrow[s, sl] = ((hv >> (PAIR_SH + 1)) << PAIR_SH) + (hv & (PAIR_BN - 1))
                trow[s, sl] = ((tv >> (PAIR_SH + 1)) << PAIR_SH) + (tv & (PAIR_BN - 1))

            pltpu.async_copy(ent_hbm.at[hrow.at[s]], hrows.at[s], sems[s])
            pltpu.async_copy(ent_hbm.at[trow.at[s]], trows.at[s], sems[s])
            pltpu.async_copy(rn_hbm.at[rids.at[s]], rn_rows.at[s], sems[s])

        def drain(s):
            pltpu.make_async_copy(
                ent_hbm.at[pl.ds(0, C)], hrows.at[s], sems[s]).wait()
            pltpu.make_async_copy(
                ent_hbm.at[pl.ds(0, C)], trows.at[s], sems[s]).wait()
            pltpu.make_async_copy(
                rn_hbm.at[pl.ds(0, C)], rn_rows.at[s], sems[s]).wait()

        fire(0, 0)
        for ch in range(NCH):
            s = ch & 1
            if ch + 1 < NCH:
                fire(ch + 1, (ch + 1) & 1)
            drain(s)

            @pl.loop(0, G, unroll=2)
            def _group(g, ch=ch, s=s):
                row = g * L + lane
                sl = pl.ds(g * L, L)
                cdh = ((hids[s, sl] >> PAIR_SH) & 1) * DIM
                cdt = ((tids[s, sl] >> PAIR_SH) & 1) * DIM
                cdn = jnp.full((L,), DIM, jnp.int32)
                un = jnp.zeros((L,), jnp.float32)
                nn = jnp.zeros((L,), jnp.float32)
                for d in range(DIM):
                    hv = plsc.load_gather(hrows.at[s], [row, cdh])
                    tv = plsc.load_gather(trows.at[s], [row, cdt])
                    nv = plsc.load_gather(rn_rows.at[s], [row, cdn])
                    uv = hv - tv
                    u_scr[d] = uv
                    n_scr[d] = nv
                    un = un + uv * nv
                    nn = nn + nv * nv
                    if d + 1 < DIM:
                        cdh = cdh + 1
                        cdt = cdt + 1
                        cdn = cdn + 1
                gamma = un / jnp.maximum(nn, 1e-24)
                cdr = jnp.zeros((L,), jnp.int32)
                acc = jnp.zeros((L,), jnp.float32)
                for d in range(DIM):
                    rv = plsc.load_gather(rn_rows.at[s], [row, cdr])
                    acc = acc + jnp.abs(u_scr[d] + rv - gamma * n_scr[d])
                    if d + 1 < DIM:
                        cdr = cdr + 1
                outv[pl.ds(ch * C + g * L, L)] = acc

        pltpu.sync_copy(outv, out_hbm.at[pl.ds(wid * PW, PW)])

    return _k(h_ids, r_ids, t_ids, ent_pair, rn_table)


def kernel(h_ids, r_ids, t_ids, entity_emb, relation_emb, normal_vec):
    ent_pair = _pair_rows_tc(entity_emb.T)
    rn_table = jnp.concatenate([relation_emb, normal_vec], axis=1)
    return _transh_sc(h_ids, r_ids, t_ids, ent_pair, rn_table)
